# baseline (speedup 1.0000x reference)
import jax
import jax.numpy as jnp
from jax.experimental import pallas as pl
from jax.experimental.pallas import tpu as pltpu

MAXLEN = 8192
OUTPUT_DIM = 2048
_CHUNK = 256
_NCHUNK = MAXLEN // _CHUNK
_NBUF = 16


def _copy_ring(table_ref, out_ref, bufs, in_sems, out_sems):
    def cin(i):
        return pltpu.make_async_copy(
            table_ref.at[pl.ds(i * _CHUNK, _CHUNK)], bufs.at[i % _NBUF],
            in_sems.at[i % _NBUF])

    def cout(i):
        return pltpu.make_async_copy(
            bufs.at[i % _NBUF], out_ref.at[pl.ds(i * _CHUNK, _CHUNK)],
            out_sems.at[i % _NBUF])

    for i in range(_NBUF):
        cin(i).start()
    for i in range(_NCHUNK):
        cin(i).wait()
        cout(i).start()
        if i + _NBUF < _NCHUNK:
            cout(i).wait()
            cin(i + _NBUF).start()
    for i in range(_NCHUNK - _NBUF, _NCHUNK):
        cout(i).wait()


def kernel(inputs, table):
    del inputs
    out = pl.pallas_call(
        _copy_ring,
        in_specs=[pl.BlockSpec(memory_space=pl.ANY)],
        out_specs=pl.BlockSpec(memory_space=pl.ANY),
        out_shape=jax.ShapeDtypeStruct((MAXLEN, OUTPUT_DIM), table.dtype),
        scratch_shapes=[
            pltpu.VMEM((_NBUF, _CHUNK, OUTPUT_DIM), jnp.float32),
            pltpu.SemaphoreType.DMA((_NBUF,)),
            pltpu.SemaphoreType.DMA((_NBUF,)),
        ],
    )(table)
    return out[None]
